# Initial kernel scaffold; baseline (speedup 1.0000x reference)
import jax
import jax.numpy as jnp
from jax.experimental import pallas as pl


def kernel(x, router_w, gate_w, up_w, down_w):
    raise NotImplementedError("write your pallas kernel here")


# speedup vs baseline: 1.3713x; 1.3713x over previous
import functools

import jax
import jax.numpy as jnp
from jax import lax
from jax.experimental import pallas as pl
from jax.experimental.pallas import tpu as pltpu
from jax.experimental.pallas import tpu_sc as plsc

T = 2048
H = 1024
F = 4096
E = 8
K = 2
P = T * K
R = 128
NB = P // R + E
P_PAD = NB * R
FT = 1024
NF = F // FT

NC = 2
NS = 16
NW = NC * NS


def _router_body(x_ref, w_ref, logits_ref, i1_ref, i2_ref, w1_ref, w2_ref):
    logits = jnp.dot(x_ref[...], w_ref[...], preferred_element_type=jnp.float32)
    logits_ref[...] = logits
    probs = jax.nn.softmax(logits, axis=-1)
    eidx = lax.broadcasted_iota(jnp.int32, (T, E), 1)
    m1 = jnp.max(probs, axis=1, keepdims=True)
    i1 = jnp.min(jnp.where(probs == m1, eidx, E), axis=1, keepdims=True)
    masked = jnp.where(eidx == i1, -1.0, probs)
    m2 = jnp.max(masked, axis=1, keepdims=True)
    i2 = jnp.min(jnp.where(masked == m2, eidx, E), axis=1, keepdims=True)
    s = m1 + m2
    i1_ref[...] = i1
    i2_ref[...] = i2
    w1_ref[...] = m1 / s
    w2_ref[...] = m2 / s


def _router(flat, router_w):
    return pl.pallas_call(
        _router_body,
        out_shape=(
            jax.ShapeDtypeStruct((T, E), jnp.float32),
            jax.ShapeDtypeStruct((T, 1), jnp.int32),
            jax.ShapeDtypeStruct((T, 1), jnp.int32),
            jax.ShapeDtypeStruct((T, 1), jnp.float32),
            jax.ShapeDtypeStruct((T, 1), jnp.float32),
        ),
    )(flat, router_w)


_DISP_ROWS = P_PAD // NW
_DISP_CHUNK = 80
_DISP_ITERS = _DISP_ROWS // _DISP_CHUNK


@functools.cache
def _sc_mesh():
    return plsc.VectorSubcoreMesh(core_axis_name="c", subcore_axis_name="s")


@functools.cache
def _build_dispatch():
    @functools.partial(
        pl.kernel,
        mesh=_sc_mesh(),
        out_type=jax.ShapeDtypeStruct((P_PAD, H), jnp.float32),
        scratch_types=[
            pltpu.VMEM((_DISP_CHUNK,), jnp.int32),
            pltpu.VMEM((_DISP_CHUNK, H), jnp.float32),
            pltpu.SemaphoreType.DMA,
        ],
    )
    def disp(x_hbm, src_hbm, xs_hbm, idx_v, rows_v, sem):
        wid = lax.axis_index("s") * NC + lax.axis_index("c")
        base = wid * _DISP_ROWS
        for c in range(_DISP_ITERS):
            off = base + c * _DISP_CHUNK
            pltpu.sync_copy(src_hbm.at[pl.ds(off, _DISP_CHUNK)], idx_v)
            pltpu.async_copy(x_hbm.at[idx_v], rows_v, sem).wait()
            pltpu.sync_copy(rows_v, xs_hbm.at[pl.ds(off, _DISP_CHUNK)])

    return disp


def _dispatch(flat, tok_src):
    return _build_dispatch()(flat, tok_src)


def _gmm_body(eid_ref, vld_ref, xs_ref, gw_ref, uw_ref, dw_ref, wr_ref, out_ref):
    j = pl.program_id(0)
    i = pl.program_id(1)

    @pl.when(vld_ref[i] == 1)
    def _():
        xb = xs_ref[...]
        g = jnp.dot(xb, gw_ref[0], preferred_element_type=jnp.float32)
        u = jnp.dot(xb, uw_ref[0], preferred_element_type=jnp.float32)
        hmid = g / (1.0 + jnp.exp(-g)) * u
        partial = jnp.dot(hmid, dw_ref[0], preferred_element_type=jnp.float32)
        rows = pl.ds(i * R, R)

        @pl.when(j == 0)
        def _():
            out_ref[rows, :] = partial

        @pl.when(j > 0)
        def _():
            out_ref[rows, :] = out_ref[rows, :] + partial

        @pl.when(j == NF - 1)
        def _():
            out_ref[rows, :] = out_ref[rows, :] * wr_ref[...]


def _gmm(eid, valid, xs, gate_w, up_w, down_w, wrow):
    grid_spec = pltpu.PrefetchScalarGridSpec(
        num_scalar_prefetch=2,
        grid=(NF, NB),
        in_specs=[
            pl.BlockSpec((R, H), lambda j, i, eid, vld: (i, 0)),
            pl.BlockSpec((1, H, FT), lambda j, i, eid, vld: (eid[i], 0, j)),
            pl.BlockSpec((1, H, FT), lambda j, i, eid, vld: (eid[i], 0, j)),
            pl.BlockSpec((1, FT, H), lambda j, i, eid, vld: (eid[i], j, 0)),
            pl.BlockSpec((R, 1), lambda j, i, eid, vld: (i, 0)),
        ],
        out_specs=pl.BlockSpec((P_PAD, H), lambda j, i, eid, vld: (0, 0)),
    )
    return pl.pallas_call(
        _gmm_body,
        grid_spec=grid_spec,
        out_shape=jax.ShapeDtypeStruct((P_PAD, H), jnp.float32),
        compiler_params=pltpu.CompilerParams(
            dimension_semantics=("arbitrary", "arbitrary"),
        ),
    )(eid, valid, xs, gate_w, up_w, down_w, wrow)


_COMB_ROWS = T // NW
_COMB_CHUNK = 32
_COMB_ITERS = _COMB_ROWS // _COMB_CHUNK
_VECS = _COMB_CHUNK * H // 16


@functools.cache
def _build_combine():
    @functools.partial(
        pl.kernel,
        mesh=_sc_mesh(),
        out_type=jax.ShapeDtypeStruct((T, H), jnp.float32),
        scratch_types=[
            pltpu.VMEM((_COMB_CHUNK,), jnp.int32),
            pltpu.VMEM((_COMB_CHUNK,), jnp.int32),
            pltpu.VMEM((_COMB_CHUNK, H), jnp.float32),
            pltpu.VMEM((_COMB_CHUNK, H), jnp.float32),
            pltpu.SemaphoreType.DMA,
            pltpu.SemaphoreType.DMA,
        ],
    )
    def comb(ys_hbm, p0_hbm, p1_hbm, out_hbm, i0_v, i1_v, a_v, b_v, s0, s1):
        wid = lax.axis_index("s") * NC + lax.axis_index("c")
        base = wid * _COMB_ROWS
        for c in range(_COMB_ITERS):
            off = base + c * _COMB_CHUNK
            pltpu.sync_copy(p0_hbm.at[pl.ds(off, _COMB_CHUNK)], i0_v)
            pltpu.sync_copy(p1_hbm.at[pl.ds(off, _COMB_CHUNK)], i1_v)
            cp0 = pltpu.async_copy(ys_hbm.at[i0_v], a_v, s0)
            cp1 = pltpu.async_copy(ys_hbm.at[i1_v], b_v, s1)
            cp0.wait()
            cp1.wait()

            def _add(k, _):
                r = k // (H // 16)
                col = (k % (H // 16)) * 16
                sl = pl.ds(col, 16)
                a_v[r, sl] = a_v[r, sl] + b_v[r, sl]
                return _

            lax.fori_loop(0, _VECS, _add, None)
            pltpu.sync_copy(a_v, out_hbm.at[pl.ds(off, _COMB_CHUNK)])

    return comb


def _combine(ys, pos0, pos1):
    return _build_combine()(ys, pos0, pos1)


def kernel(x, router_w, gate_w, up_w, down_w):
    flat = x.reshape(T, H)
    logits, i1, i2, w1, w2 = _router(flat, router_w)

    e_flat = jnp.concatenate([i1, i2], axis=1).reshape(P)
    w_flat = jnp.concatenate([w1, w2], axis=1).reshape(P)
    order = jnp.argsort(e_flat, stable=True)
    sorted_e = e_flat[order]
    counts = jnp.bincount(e_flat, length=E)
    group_start = jnp.concatenate(
        [jnp.zeros((1,), jnp.int32), jnp.cumsum(counts)[:-1].astype(jnp.int32)])
    padded = ((counts + R - 1) // R) * R
    padded_off = jnp.concatenate(
        [jnp.zeros((1,), jnp.int32), jnp.cumsum(padded)[:-1].astype(jnp.int32)])
    ranks = jnp.arange(P, dtype=jnp.int32) - group_start[sorted_e]
    dest = padded_off[sorted_e] + ranks
    tok = (order // K).astype(jnp.int32)
    tok_src = jnp.zeros((P_PAD,), jnp.int32).at[dest].set(tok)
    wrow = jnp.zeros((P_PAD,), jnp.float32).at[dest].set(w_flat[order])
    wrow = wrow.reshape(P_PAD, 1)
    posq = jnp.zeros((P,), jnp.int32).at[order].set(dest)
    pos0 = posq[0::2]
    pos1 = posq[1::2]
    nb_e = (padded // R).astype(jnp.int32)
    eid = jnp.repeat(jnp.arange(E, dtype=jnp.int32), nb_e,
                     total_repeat_length=NB)
    valid = (jnp.arange(NB) < jnp.sum(nb_e)).astype(jnp.int32)

    xs = _dispatch(flat, tok_src)
    ys = _gmm(eid, valid, xs, gate_w, up_w, down_w, wrow)
    out = _combine(ys, pos0, pos1)
    return out.reshape(1, T, H), logits
